# baseline (speedup 1.0000x reference)
import functools

import jax
import jax.numpy as jnp
from jax import lax
from jax.experimental import pallas as pl
from jax.experimental.pallas import tpu as pltpu

_A = 16
_ADIM = 8
_NBS = 16
_HID = 256
_HEADS = 4
_OUTC = _HID // _HEADS
_LAT = 64
_NEG = -1e30


def _leaky(x):
    return jnp.where(x >= 0, x, 0.2 * x)


def _elu(x):
    return jnp.where(x > 0, x, jnp.exp(x) - 1.0)


def _gat_dst(als, ald, h3, sel, j):
    g = als.shape[0]
    dj = ald[:, j:j + 1, :]
    e = _leaky(als + dj)
    ii = lax.broadcasted_iota(jnp.int32, (g, _A, 1), 1)
    e = jnp.where(ii == j, _NEG, e)
    ex = jnp.exp(e)
    den = jnp.sum(ex, axis=1, keepdims=True)
    alpha = ex * (1.0 / (den + 1e-16))
    af = jnp.dot(alpha.reshape(g * _A, _HEADS), sel,
                 preferred_element_type=jnp.float32).reshape(g, _A, _HID)
    return jnp.sum(af * h3, axis=1)


def _gat_all(s_all, ald, h3):
    g = s_all.shape[0]
    aldT = jnp.swapaxes(ald, 1, 2)
    d_all = jnp.concatenate(
        [aldT[:, h:h + 1, :] for h in range(_HEADS)], axis=2)
    e = _leaky(s_all + d_all)
    ii = lax.broadcasted_iota(jnp.int32, (g, _A, _HEADS * _A), 1)
    jj = lax.broadcasted_iota(jnp.int32, (g, _A, _HEADS * _A), 2)
    e = jnp.where(ii == jj % _A, _NEG, e)
    ex = jnp.exp(e)
    den = jnp.sum(ex, axis=1, keepdims=True)
    alpha = ex * (1.0 / (den + 1e-16))
    parts = [
        lax.dot_general(
            alpha[:, :, h * _A:(h + 1) * _A],
            h3[:, :, h * _OUTC:(h + 1) * _OUTC],
            (((1,), (1,)), ((0,), (0,))),
            preferred_element_type=jnp.float32)
        for h in range(_HEADS)
    ]
    return jnp.concatenate(parts, axis=2)


def _fused(sig_ref, act_ref, w0s_ref, w0a_ref, a0_ref, b0_ref,
           w1_ref, a1_ref, b1_ref, sel_ref, wv_ref, bv_ref,
           wo_ref, bo_ref, wm_ref, bm_ref, wa_ref, ba_ref, out_ref):
    g = sig_ref.shape[0]
    sel = sel_ref[:]

    h0 = jnp.dot(act_ref[:], w0a_ref[:],
                 preferred_element_type=jnp.float32)
    sig_h = jnp.dot(sig_ref[:], w0s_ref[:],
                    preferred_element_type=jnp.float32)
    h0 = h0.reshape(g, _A, _HID)
    node = lax.broadcasted_iota(jnp.int32, (g, _A, 1), 1)
    h0 = h0 + jnp.where(node == 0, sig_h[:, None, :], 0.0)

    al = jnp.dot(h0.reshape(g * _A, _HID), a0_ref[:],
                 preferred_element_type=jnp.float32
                 ).reshape(g, _A, _HEADS * _A + 2 * _HEADS)
    hg = _gat_all(al[:, :, :_HEADS * _A],
                  al[:, :, _HEADS * _A:_HEADS * _A + _HEADS], h0)
    hg = _elu(hg + b0_ref[:])

    h1 = jnp.dot(hg.reshape(g * _A, _HID), w1_ref[:],
                 preferred_element_type=jnp.float32)
    al1 = jnp.dot(h1, a1_ref[:],
                  preferred_element_type=jnp.float32
                  ).reshape(g, _A, 2 * _HEADS)
    h1 = h1.reshape(g, _A, _HID)
    feat = _gat_dst(al1[:, :, :_HEADS], al1[:, :, _HEADS:], h1, sel, 0)
    feat = _elu(feat + b1_ref[:])

    v = jnp.dot(feat, wv_ref[:], preferred_element_type=jnp.float32) + bv_ref[:]
    ao = jnp.dot(v, wo_ref[:], preferred_element_type=jnp.float32) + bo_ref[:]
    mean = jnp.dot(ao, wm_ref[:], preferred_element_type=jnp.float32) + bm_ref[:]
    out_ref[:] = jnp.dot(mean, wa_ref[:],
                         preferred_element_type=jnp.float32) + ba_ref[:]


def _att_mat(a):
    eye = jnp.eye(_HEADS, dtype=jnp.float32)
    return (a[:, :, None] * eye[:, None, :]).reshape(_HID, _HEADS)


def kernel(signals, neighbor_actions, agent_id, W0, att_src0, att_dst0, bias0,
           W1, att_src1, att_dst1, bias1, in_proj_w, in_proj_b, out_proj_w,
           out_proj_b, Wm, bm, Wa, ba):
    b = signals.shape[0]
    g = 256 if b % 256 == 0 else b

    idx = jnp.arange(_A, dtype=jnp.int32)
    order = jnp.argsort(jnp.where(idx == agent_id, -1, idx), stable=True)
    act = jnp.take(neighbor_actions.reshape(b, _A, _ADIM), order, axis=1)
    act2d = act.reshape(b * _A, _ADIM)

    def att_wide(a_src, a_dst):
        s, d = _att_mat(a_src), _att_mat(a_dst)
        return jnp.concatenate([jnp.repeat(s, _A, axis=1), d, s], axis=1)

    a0 = att_wide(att_src0, att_dst0)
    a1 = jnp.concatenate([_att_mat(att_src1), _att_mat(att_dst1)], axis=1)
    eye = jnp.eye(_HEADS, dtype=jnp.float32)
    sel = (eye[:, :, None] * jnp.ones((1, 1, _OUTC), jnp.float32)
           ).reshape(_HEADS, _HID)
    full = lambda shape: pl.BlockSpec(shape, lambda i: (0, 0))
    row = lambda r, c: pl.BlockSpec((r, c), lambda i: (i, 0))

    return pl.pallas_call(
        _fused,
        grid=(b // g,),
        in_specs=[
            row(g, _NBS),
            row(g * _A, _ADIM),
            full((_NBS, _HID)),
            full((_ADIM, _HID)),
            full((_HID, _HEADS * _A + 2 * _HEADS)),
            full((1, _HID)),
            full((_HID, _HID)),
            full((_HID, 2 * _HEADS)),
            full((1, _HID)),
            full((_HEADS, _HID)),
            full((_HID, _HID)),
            full((1, _HID)),
            full((_HID, _HID)),
            full((1, _HID)),
            full((_HID, _LAT)),
            full((1, _LAT)),
            full((_LAT, _A * _ADIM)),
            full((1, _A * _ADIM)),
        ],
        out_specs=row(g, _A * _ADIM),
        out_shape=jax.ShapeDtypeStruct((b, _A * _ADIM), jnp.float32),
        compiler_params=pltpu.CompilerParams(
            dimension_semantics=("parallel",)),
    )(
        signals, act2d, W0[:_NBS], W0[_NBS:], a0, bias0[None, :],
        W1, a1, bias1[None, :], sel,
        in_proj_w[2 * _HID:].T, in_proj_b[None, 2 * _HID:],
        out_proj_w.T, out_proj_b[None, :],
        Wm.T, bm[None, :], Wa.T, ba[None, :],
    )


# speedup vs baseline: 1.0482x; 1.0482x over previous
import functools

import jax
import jax.numpy as jnp
from jax import lax
from jax.experimental import pallas as pl
from jax.experimental.pallas import tpu as pltpu

_A = 16
_ADIM = 8
_NBS = 16
_HID = 256
_HEADS = 4
_OUTC = _HID // _HEADS
_LAT = 64
_NEG = -1e30


def _leaky(x):
    return jnp.where(x >= 0, x, 0.2 * x)


def _elu(x):
    return jnp.where(x > 0, x, jnp.exp(x) - 1.0)


def _gat_dst(als, ald, h3, j):
    g = als.shape[0]
    dj = ald[:, j:j + 1, :]
    e = _leaky(als + dj)
    ii = lax.broadcasted_iota(jnp.int32, (g, _A, 1), 1)
    e = jnp.where(ii == j, _NEG, e)
    ex = jnp.exp(e)
    den = jnp.sum(ex, axis=1, keepdims=True)
    alpha = ex * (1.0 / (den + 1e-16))
    agg = lax.dot_general(alpha, h3, (((1,), (1,)), ((0,), (0,))),
                          preferred_element_type=jnp.float32)
    return jnp.concatenate(
        [agg[:, h, h * _OUTC:(h + 1) * _OUTC] for h in range(_HEADS)],
        axis=1)


def _gat_all(s_all, ald, h3):
    g = s_all.shape[0]
    aldT = jnp.swapaxes(ald, 1, 2)
    d_all = jnp.concatenate(
        [aldT[:, h:h + 1, :] for h in range(_HEADS)], axis=2)
    e = _leaky(s_all + d_all)
    ii = lax.broadcasted_iota(jnp.int32, (g, _A, _HEADS * _A), 1)
    jj = lax.broadcasted_iota(jnp.int32, (g, _A, _HEADS * _A), 2)
    e = jnp.where(ii == jj % _A, _NEG, e)
    ex = jnp.exp(e)
    den = jnp.sum(ex, axis=1, keepdims=True)
    alpha = ex * (1.0 / (den + 1e-16))
    parts = [
        lax.dot_general(
            alpha[:, :, h * _A:(h + 1) * _A],
            h3[:, :, h * _OUTC:(h + 1) * _OUTC],
            (((1,), (1,)), ((0,), (0,))),
            preferred_element_type=jnp.float32)
        for h in range(_HEADS)
    ]
    return jnp.concatenate(parts, axis=2)


def _fused(sig_ref, act_ref, w0s_ref, w0a_ref, a0_ref, b0_ref,
           w1_ref, a1_ref, b1_ref, wv_ref, bv_ref,
           wo_ref, bo_ref, wm_ref, bm_ref, wa_ref, ba_ref, out_ref):
    g = sig_ref.shape[0]

    h0 = jnp.dot(act_ref[:], w0a_ref[:],
                 preferred_element_type=jnp.float32)
    sig_h = jnp.dot(sig_ref[:], w0s_ref[:],
                    preferred_element_type=jnp.float32)
    h0 = h0.reshape(g, _A, _HID)
    node = lax.broadcasted_iota(jnp.int32, (g, _A, 1), 1)
    h0 = h0 + jnp.where(node == 0, sig_h[:, None, :], 0.0)

    al = jnp.dot(h0.reshape(g * _A, _HID), a0_ref[:],
                 preferred_element_type=jnp.float32
                 ).reshape(g, _A, _HEADS * _A + 2 * _HEADS)
    hg = _gat_all(al[:, :, :_HEADS * _A],
                  al[:, :, _HEADS * _A:_HEADS * _A + _HEADS], h0)
    hg = _elu(hg + b0_ref[:])

    h1 = jnp.dot(hg.reshape(g * _A, _HID), w1_ref[:],
                 preferred_element_type=jnp.float32)
    al1 = jnp.dot(h1, a1_ref[:],
                  preferred_element_type=jnp.float32
                  ).reshape(g, _A, 2 * _HEADS)
    h1 = h1.reshape(g, _A, _HID)
    feat = _gat_dst(al1[:, :, :_HEADS], al1[:, :, _HEADS:], h1, 0)
    feat = _elu(feat + b1_ref[:])

    v = jnp.dot(feat, wv_ref[:], preferred_element_type=jnp.float32) + bv_ref[:]
    ao = jnp.dot(v, wo_ref[:], preferred_element_type=jnp.float32) + bo_ref[:]
    mean = jnp.dot(ao, wm_ref[:], preferred_element_type=jnp.float32) + bm_ref[:]
    out_ref[:] = jnp.dot(mean, wa_ref[:],
                         preferred_element_type=jnp.float32) + ba_ref[:]


def _att_mat(a):
    eye = jnp.eye(_HEADS, dtype=jnp.float32)
    return (a[:, :, None] * eye[:, None, :]).reshape(_HID, _HEADS)


def kernel(signals, neighbor_actions, agent_id, W0, att_src0, att_dst0, bias0,
           W1, att_src1, att_dst1, bias1, in_proj_w, in_proj_b, out_proj_w,
           out_proj_b, Wm, bm, Wa, ba):
    b = signals.shape[0]
    g = 256 if b % 256 == 0 else b

    idx = jnp.arange(_A, dtype=jnp.int32)
    order = jnp.argsort(jnp.where(idx == agent_id, -1, idx), stable=True)
    act = jnp.take(neighbor_actions.reshape(b, _A, _ADIM), order, axis=1)
    act2d = act.reshape(b * _A, _ADIM)

    def att_wide(a_src, a_dst):
        s, d = _att_mat(a_src), _att_mat(a_dst)
        return jnp.concatenate([jnp.repeat(s, _A, axis=1), d, s], axis=1)

    a0 = att_wide(att_src0, att_dst0)
    a1 = jnp.concatenate([_att_mat(att_src1), _att_mat(att_dst1)], axis=1)
    full = lambda shape: pl.BlockSpec(shape, lambda i: (0, 0))
    row = lambda r, c: pl.BlockSpec((r, c), lambda i: (i, 0))

    return pl.pallas_call(
        _fused,
        grid=(b // g,),
        in_specs=[
            row(g, _NBS),
            row(g * _A, _ADIM),
            full((_NBS, _HID)),
            full((_ADIM, _HID)),
            full((_HID, _HEADS * _A + 2 * _HEADS)),
            full((1, _HID)),
            full((_HID, _HID)),
            full((_HID, 2 * _HEADS)),
            full((1, _HID)),
            full((_HID, _HID)),
            full((1, _HID)),
            full((_HID, _HID)),
            full((1, _HID)),
            full((_HID, _LAT)),
            full((1, _LAT)),
            full((_LAT, _A * _ADIM)),
            full((1, _A * _ADIM)),
        ],
        out_specs=row(g, _A * _ADIM),
        out_shape=jax.ShapeDtypeStruct((b, _A * _ADIM), jnp.float32),
        compiler_params=pltpu.CompilerParams(
            dimension_semantics=("parallel",)),
    )(
        signals, act2d, W0[:_NBS], W0[_NBS:], a0, bias0[None, :],
        W1, a1, bias1[None, :],
        in_proj_w[2 * _HID:].T, in_proj_b[None, 2 * _HID:],
        out_proj_w.T, out_proj_b[None, :],
        Wm.T, bm[None, :], Wa.T, ba[None, :],
    )


# speedup vs baseline: 1.0658x; 1.0168x over previous
import functools

import jax
import jax.numpy as jnp
from jax import lax
from jax.experimental import pallas as pl
from jax.experimental.pallas import tpu as pltpu

_A = 16
_ADIM = 8
_NBS = 16
_HID = 256
_HEADS = 4
_OUTC = _HID // _HEADS
_LAT = 64
_NEG = -1e30


def _leaky(x):
    return jnp.where(x >= 0, x, 0.2 * x)


def _elu(x):
    return jnp.where(x > 0, x, jnp.exp(x) - 1.0)


def _gat_dst(als, ald, h3, j):
    g = als.shape[0]
    dj = ald[:, j:j + 1, :]
    e = _leaky(als + dj)
    ii = lax.broadcasted_iota(jnp.int32, (g, _A, 1), 1)
    e = jnp.where(ii == j, _NEG, e)
    ex = jnp.exp(e)
    den = jnp.sum(ex, axis=1, keepdims=True)
    alpha = ex * (1.0 / (den + 1e-16))
    agg = lax.dot_general(alpha, h3, (((1,), (1,)), ((0,), (0,))),
                          preferred_element_type=jnp.float32)
    return jnp.concatenate(
        [agg[:, h, h * _OUTC:(h + 1) * _OUTC] for h in range(_HEADS)],
        axis=1)


def _gat_all(s_all, ald, h3):
    g = s_all.shape[0]
    aldT = jnp.swapaxes(ald, 1, 2)
    d_all = jnp.concatenate(
        [aldT[:, h:h + 1, :] for h in range(_HEADS)], axis=2)
    e = _leaky(s_all + d_all)
    ii = lax.broadcasted_iota(jnp.int32, (g, _A, _HEADS * _A), 1)
    jj = lax.broadcasted_iota(jnp.int32, (g, _A, _HEADS * _A), 2)
    e = jnp.where(ii == jj % _A, _NEG, e)
    ex = jnp.exp(e)
    den = jnp.sum(ex, axis=1, keepdims=True)
    alpha = ex * (1.0 / (den + 1e-16))
    parts = [
        lax.dot_general(
            alpha[:, :, h * _A:(h + 1) * _A],
            h3[:, :, h * _OUTC:(h + 1) * _OUTC],
            (((1,), (1,)), ((0,), (0,))),
            preferred_element_type=jnp.float32)
        for h in range(_HEADS)
    ]
    return jnp.concatenate(parts, axis=2)


def _fused(sig_ref, act_ref, w0s_ref, w0a_ref, a0_ref, b0_ref,
           w1_ref, a1_ref, b1_ref, wv_ref, bv_ref,
           wo_ref, bo_ref, wm_ref, bm_ref, wa_ref, ba_ref, out_ref):
    g = sig_ref.shape[0] // 2

    def body(sig, act):
        h0 = jnp.dot(act, w0a_ref[:],
                     preferred_element_type=jnp.float32)
        sig_h = jnp.dot(sig, w0s_ref[:],
                        preferred_element_type=jnp.float32)
        h0 = h0.reshape(g, _A, _HID)
        node = lax.broadcasted_iota(jnp.int32, (g, _A, 1), 1)
        h0 = h0 + jnp.where(node == 0, sig_h[:, None, :], 0.0)

        al = jnp.dot(h0.reshape(g * _A, _HID), a0_ref[:],
                     preferred_element_type=jnp.float32
                     ).reshape(g, _A, _HEADS * _A + 2 * _HEADS)
        hg = _gat_all(al[:, :, :_HEADS * _A],
                      al[:, :, _HEADS * _A:_HEADS * _A + _HEADS], h0)
        hg = _elu(hg + b0_ref[:])

        h1 = jnp.dot(hg.reshape(g * _A, _HID), w1_ref[:],
                     preferred_element_type=jnp.float32)
        al1 = jnp.dot(h1, a1_ref[:],
                      preferred_element_type=jnp.float32
                      ).reshape(g, _A, 2 * _HEADS)
        h1 = h1.reshape(g, _A, _HID)
        feat = _gat_dst(al1[:, :, :_HEADS], al1[:, :, _HEADS:], h1, 0)
        feat = _elu(feat + b1_ref[:])

        v = jnp.dot(feat, wv_ref[:],
                    preferred_element_type=jnp.float32) + bv_ref[:]
        ao = jnp.dot(v, wo_ref[:],
                     preferred_element_type=jnp.float32) + bo_ref[:]
        mean = jnp.dot(ao, wm_ref[:],
                       preferred_element_type=jnp.float32) + bm_ref[:]
        return jnp.dot(mean, wa_ref[:],
                       preferred_element_type=jnp.float32) + ba_ref[:]

    out_ref[:g] = body(sig_ref[:g], act_ref[:g * _A])
    out_ref[g:] = body(sig_ref[g:], act_ref[g * _A:])


def _att_mat(a):
    eye = jnp.eye(_HEADS, dtype=jnp.float32)
    return (a[:, :, None] * eye[:, None, :]).reshape(_HID, _HEADS)


def kernel(signals, neighbor_actions, agent_id, W0, att_src0, att_dst0, bias0,
           W1, att_src1, att_dst1, bias1, in_proj_w, in_proj_b, out_proj_w,
           out_proj_b, Wm, bm, Wa, ba):
    b = signals.shape[0]
    g = 256 if b % 256 == 0 else b

    idx = jnp.arange(_A, dtype=jnp.int32)
    order = jnp.argsort(jnp.where(idx == agent_id, -1, idx), stable=True)
    act = jnp.take(neighbor_actions.reshape(b, _A, _ADIM), order, axis=1)
    act2d = act.reshape(b * _A, _ADIM)

    def att_wide(a_src, a_dst):
        s, d = _att_mat(a_src), _att_mat(a_dst)
        return jnp.concatenate([jnp.repeat(s, _A, axis=1), d, s], axis=1)

    a0 = att_wide(att_src0, att_dst0)
    a1 = jnp.concatenate([_att_mat(att_src1), _att_mat(att_dst1)], axis=1)
    full = lambda shape: pl.BlockSpec(shape, lambda i: (0, 0))
    row = lambda r, c: pl.BlockSpec((r, c), lambda i: (i, 0))

    return pl.pallas_call(
        _fused,
        grid=(b // g,),
        in_specs=[
            row(g, _NBS),
            row(g * _A, _ADIM),
            full((_NBS, _HID)),
            full((_ADIM, _HID)),
            full((_HID, _HEADS * _A + 2 * _HEADS)),
            full((1, _HID)),
            full((_HID, _HID)),
            full((_HID, 2 * _HEADS)),
            full((1, _HID)),
            full((_HID, _HID)),
            full((1, _HID)),
            full((_HID, _HID)),
            full((1, _HID)),
            full((_HID, _LAT)),
            full((1, _LAT)),
            full((_LAT, _A * _ADIM)),
            full((1, _A * _ADIM)),
        ],
        out_specs=row(g, _A * _ADIM),
        out_shape=jax.ShapeDtypeStruct((b, _A * _ADIM), jnp.float32),
        compiler_params=pltpu.CompilerParams(
            dimension_semantics=("parallel",)),
    )(
        signals, act2d, W0[:_NBS], W0[_NBS:], a0, bias0[None, :],
        W1, a1, bias1[None, :],
        in_proj_w[2 * _HID:].T, in_proj_b[None, 2 * _HID:],
        out_proj_w.T, out_proj_b[None, :],
        Wm.T, bm[None, :], Wa.T, ba[None, :],
    )


# speedup vs baseline: 1.1086x; 1.0402x over previous
import functools

import jax
import jax.numpy as jnp
from jax import lax
from jax.experimental import pallas as pl
from jax.experimental.pallas import tpu as pltpu

_A = 16
_ADIM = 8
_NBS = 16
_HID = 256
_HEADS = 4
_OUTC = _HID // _HEADS
_LAT = 64
_NEG = -1e30


def _dot_t(x, w):
    return lax.dot_general(x, w, (((1,), (1,)), ((), ())),
                           preferred_element_type=jnp.float32)


def _leaky(x):
    return jnp.where(x >= 0, x, 0.2 * x)


def _elu(x):
    return jnp.where(x > 0, x, jnp.exp(x) - 1.0)


def _gat_dst(als, ald, h3, j):
    g = als.shape[0]
    dj = ald[:, j:j + 1, :]
    e = _leaky(als + dj)
    ii = lax.broadcasted_iota(jnp.int32, (g, _A, 1), 1)
    e = jnp.where(ii == j, _NEG, e)
    ex = jnp.exp(e)
    den = jnp.sum(ex, axis=1, keepdims=True)
    alpha = ex * (1.0 / (den + 1e-16))
    agg = lax.dot_general(alpha, h3, (((1,), (1,)), ((0,), (0,))),
                          preferred_element_type=jnp.float32)
    return jnp.concatenate(
        [agg[:, h, h * _OUTC:(h + 1) * _OUTC] for h in range(_HEADS)],
        axis=1)


def _gat_all(s_all, ald, h3):
    g = s_all.shape[0]
    aldT = jnp.swapaxes(ald, 1, 2)
    d_all = jnp.concatenate(
        [aldT[:, h:h + 1, :] for h in range(_HEADS)], axis=2)
    e = _leaky(s_all + d_all)
    ii = lax.broadcasted_iota(jnp.int32, (g, _A, _HEADS * _A), 1)
    jj = lax.broadcasted_iota(jnp.int32, (g, _A, _HEADS * _A), 2)
    e = jnp.where(ii == jj % _A, _NEG, e)
    ex = jnp.exp(e)
    den = jnp.sum(ex, axis=1, keepdims=True)
    alpha = ex * (1.0 / (den + 1e-16))
    parts = [
        lax.dot_general(
            alpha[:, :, h * _A:(h + 1) * _A],
            h3[:, :, h * _OUTC:(h + 1) * _OUTC],
            (((1,), (1,)), ((0,), (0,))),
            preferred_element_type=jnp.float32)
        for h in range(_HEADS)
    ]
    return jnp.concatenate(parts, axis=2)


def _fused(sig_ref, act_ref, w0_ref, a0_ref, b0_ref,
           w1_ref, a1_ref, b1_ref, wv_ref, bv_ref,
           wo_ref, bo_ref, wm_ref, bm_ref, wa_ref, ba_ref, out_ref):
    g = sig_ref.shape[0] // 2

    def body(sig, act):
        h0 = jnp.dot(act, w0_ref[_NBS:],
                     preferred_element_type=jnp.float32)
        sig_h = jnp.dot(sig, w0_ref[:_NBS],
                        preferred_element_type=jnp.float32)
        h0 = h0.reshape(g, _A, _HID)
        node = lax.broadcasted_iota(jnp.int32, (g, _A, 1), 1)
        h0 = h0 + jnp.where(node == 0, sig_h[:, None, :], 0.0)

        al = jnp.dot(h0.reshape(g * _A, _HID), a0_ref[:],
                     preferred_element_type=jnp.float32
                     ).reshape(g, _A, _HEADS * _A + 2 * _HEADS)
        hg = _gat_all(al[:, :, :_HEADS * _A],
                      al[:, :, _HEADS * _A:_HEADS * _A + _HEADS], h0)
        hg = _elu(hg + b0_ref[:])

        h1 = jnp.dot(hg.reshape(g * _A, _HID), w1_ref[:],
                     preferred_element_type=jnp.float32)
        al1 = jnp.dot(h1, a1_ref[:],
                      preferred_element_type=jnp.float32
                      ).reshape(g, _A, 2 * _HEADS)
        h1 = h1.reshape(g, _A, _HID)
        feat = _gat_dst(al1[:, :, :_HEADS], al1[:, :, _HEADS:], h1, 0)
        feat = _elu(feat + b1_ref[:])

        v = _dot_t(feat, wv_ref[:]) + bv_ref[:]
        ao = _dot_t(v, wo_ref[:]) + bo_ref[:]
        mean = _dot_t(ao, wm_ref[:]) + bm_ref[:]
        return _dot_t(mean, wa_ref[:]) + ba_ref[:]

    out_ref[:g] = body(sig_ref[:g], act_ref[:g * _A])
    out_ref[g:] = body(sig_ref[g:], act_ref[g * _A:])


def _att_mat(a):
    eye = jnp.eye(_HEADS, dtype=jnp.float32)
    return (a[:, :, None] * eye[:, None, :]).reshape(_HID, _HEADS)


def kernel(signals, neighbor_actions, agent_id, W0, att_src0, att_dst0, bias0,
           W1, att_src1, att_dst1, bias1, in_proj_w, in_proj_b, out_proj_w,
           out_proj_b, Wm, bm, Wa, ba):
    b = signals.shape[0]
    g = 256 if b % 256 == 0 else b

    idx = jnp.arange(_A, dtype=jnp.int32)
    order = jnp.argsort(jnp.where(idx == agent_id, -1, idx), stable=True)
    act = jnp.take(neighbor_actions.reshape(b, _A, _ADIM), order, axis=1)
    act2d = act.reshape(b * _A, _ADIM)

    def att_wide(a_src, a_dst):
        s, d = _att_mat(a_src), _att_mat(a_dst)
        return jnp.concatenate([jnp.repeat(s, _A, axis=1), d, s], axis=1)

    a0 = att_wide(att_src0, att_dst0)
    a1 = jnp.concatenate([_att_mat(att_src1), _att_mat(att_dst1)], axis=1)
    full = lambda shape: pl.BlockSpec(shape, lambda i: (0, 0))
    row = lambda r, c: pl.BlockSpec((r, c), lambda i: (i, 0))

    return pl.pallas_call(
        _fused,
        grid=(b // g,),
        in_specs=[
            row(g, _NBS),
            row(g * _A, _ADIM),
            full((_NBS + _ADIM, _HID)),
            full((_HID, _HEADS * _A + 2 * _HEADS)),
            full((1, _HID)),
            full((_HID, _HID)),
            full((_HID, 2 * _HEADS)),
            full((1, _HID)),
            pl.BlockSpec((_HID, _HID), lambda i: (2, 0)),
            pl.BlockSpec((1, _HID), lambda i: (0, 2)),
            full((_HID, _HID)),
            full((1, _HID)),
            full((_LAT, _HID)),
            full((1, _LAT)),
            full((_A * _ADIM, _LAT)),
            full((1, _A * _ADIM)),
        ],
        out_specs=row(g, _A * _ADIM),
        out_shape=jax.ShapeDtypeStruct((b, _A * _ADIM), jnp.float32),
        compiler_params=pltpu.CompilerParams(
            dimension_semantics=("parallel",)),
    )(
        signals, act2d, W0, a0, bias0[None, :],
        W1, a1, bias1[None, :],
        in_proj_w, in_proj_b[None, :],
        out_proj_w, out_proj_b[None, :],
        Wm, bm[None, :], Wa, ba[None, :],
    )


# speedup vs baseline: 1.2872x; 1.1611x over previous
import functools

import jax
import jax.numpy as jnp
from jax import lax
from jax.experimental import pallas as pl
from jax.experimental.pallas import tpu as pltpu

_A = 16
_ADIM = 8
_NBS = 16
_HID = 256
_HEADS = 4
_OUTC = _HID // _HEADS
_LAT = 64
_NEG = -1e30


def _dot_t(x, w):
    return lax.dot_general(x, w, (((1,), (1,)), ((), ())),
                           preferred_element_type=jnp.float32)


def _leaky(x):
    return jnp.where(x >= 0, x, 0.2 * x)


def _elu(x):
    return jnp.where(x > 0, x, jnp.exp(x) - 1.0)


def _gat_dst(als, ald, h3, dst):
    g = als.shape[0]
    ii = lax.broadcasted_iota(jnp.int32, (g, _A, 1), 1)
    is_dst = ii == dst
    dj = jnp.sum(jnp.where(is_dst, ald, 0.0), axis=1, keepdims=True)
    e = _leaky(als + dj)
    e = jnp.where(is_dst, _NEG, e)
    ex = jnp.exp(e)
    den = jnp.sum(ex, axis=1, keepdims=True)
    alpha = ex * (1.0 / (den + 1e-16))
    agg = lax.dot_general(alpha, h3, (((1,), (1,)), ((0,), (0,))),
                          preferred_element_type=jnp.float32)
    return jnp.concatenate(
        [agg[:, h, h * _OUTC:(h + 1) * _OUTC] for h in range(_HEADS)],
        axis=1)


def _gat_all(s_all, ald, h3):
    g = s_all.shape[0]
    aldT = jnp.swapaxes(ald, 1, 2)
    d_all = jnp.concatenate(
        [aldT[:, h:h + 1, :] for h in range(_HEADS)], axis=2)
    e = _leaky(s_all + d_all)
    ii = lax.broadcasted_iota(jnp.int32, (g, _A, _HEADS * _A), 1)
    jj = lax.broadcasted_iota(jnp.int32, (g, _A, _HEADS * _A), 2)
    e = jnp.where(ii == jj % _A, _NEG, e)
    ex = jnp.exp(e)
    den = jnp.sum(ex, axis=1, keepdims=True)
    alpha = ex * (1.0 / (den + 1e-16))
    parts = [
        lax.dot_general(
            alpha[:, :, h * _A:(h + 1) * _A],
            h3[:, :, h * _OUTC:(h + 1) * _OUTC],
            (((1,), (1,)), ((0,), (0,))),
            preferred_element_type=jnp.float32)
        for h in range(_HEADS)
    ]
    return jnp.concatenate(parts, axis=2)


def _fused(aid_ref, sig_ref, act_ref, w0_ref, a0_ref, b0_ref,
           w1_ref, a1_ref, b1_ref, wv_ref, bv_ref,
           wo_ref, bo_ref, wm_ref, bm_ref, wa_ref, ba_ref, out_ref):
    g = sig_ref.shape[0] // 2
    aid = aid_ref[0]

    def body(sig, act):
        h0 = jnp.dot(act, w0_ref[_NBS:],
                     preferred_element_type=jnp.float32)
        sig_h = jnp.dot(sig, w0_ref[:_NBS],
                        preferred_element_type=jnp.float32)
        h0 = h0.reshape(g, _A, _HID)
        node = lax.broadcasted_iota(jnp.int32, (g, _A, 1), 1)
        h0 = h0 + jnp.where(node == aid, sig_h[:, None, :], 0.0)

        al = jnp.dot(h0.reshape(g * _A, _HID), a0_ref[:],
                     preferred_element_type=jnp.float32
                     ).reshape(g, _A, _HEADS * _A + 2 * _HEADS)
        hg = _gat_all(al[:, :, :_HEADS * _A],
                      al[:, :, _HEADS * _A:_HEADS * _A + _HEADS], h0)
        hg = _elu(hg + b0_ref[:])

        h1 = jnp.dot(hg.reshape(g * _A, _HID), w1_ref[:],
                     preferred_element_type=jnp.float32)
        al1 = jnp.dot(h1, a1_ref[:],
                      preferred_element_type=jnp.float32
                      ).reshape(g, _A, 2 * _HEADS)
        h1 = h1.reshape(g, _A, _HID)
        feat = _gat_dst(al1[:, :, :_HEADS], al1[:, :, _HEADS:], h1, aid)
        feat = _elu(feat + b1_ref[:])

        v = _dot_t(feat, wv_ref[:]) + bv_ref[:]
        ao = _dot_t(v, wo_ref[:]) + bo_ref[:]
        mean = _dot_t(ao, wm_ref[:]) + bm_ref[:]
        return _dot_t(mean, wa_ref[:]) + ba_ref[:]

    out_ref[:g] = body(sig_ref[:g], act_ref[:g * _A])
    out_ref[g:] = body(sig_ref[g:], act_ref[g * _A:])


def _att_mat(a):
    eye = jnp.eye(_HEADS, dtype=jnp.float32)
    return (a[:, :, None] * eye[:, None, :]).reshape(_HID, _HEADS)


def kernel(signals, neighbor_actions, agent_id, W0, att_src0, att_dst0, bias0,
           W1, att_src1, att_dst1, bias1, in_proj_w, in_proj_b, out_proj_w,
           out_proj_b, Wm, bm, Wa, ba):
    b = signals.shape[0]
    g = 256 if b % 256 == 0 else b

    act2d = neighbor_actions.reshape(b * _A, _ADIM)
    aid = jnp.asarray(agent_id, jnp.int32).reshape(1)

    def att_wide(a_src, a_dst):
        s, d = _att_mat(a_src), _att_mat(a_dst)
        return jnp.concatenate([jnp.repeat(s, _A, axis=1), d, s], axis=1)

    a0 = att_wide(att_src0, att_dst0)
    a1 = jnp.concatenate([_att_mat(att_src1), _att_mat(att_dst1)], axis=1)
    full = lambda shape: pl.BlockSpec(shape, lambda i: (0, 0))
    row = lambda r, c: pl.BlockSpec((r, c), lambda i: (i, 0))

    return pl.pallas_call(
        _fused,
        grid=(b // g,),
        in_specs=[
            pl.BlockSpec(memory_space=pltpu.SMEM),
            row(g, _NBS),
            row(g * _A, _ADIM),
            full((_NBS + _ADIM, _HID)),
            full((_HID, _HEADS * _A + 2 * _HEADS)),
            full((1, _HID)),
            full((_HID, _HID)),
            full((_HID, 2 * _HEADS)),
            full((1, _HID)),
            pl.BlockSpec((_HID, _HID), lambda i: (2, 0)),
            pl.BlockSpec((1, _HID), lambda i: (0, 2)),
            full((_HID, _HID)),
            full((1, _HID)),
            full((_LAT, _HID)),
            full((1, _LAT)),
            full((_A * _ADIM, _LAT)),
            full((1, _A * _ADIM)),
        ],
        out_specs=row(g, _A * _ADIM),
        out_shape=jax.ShapeDtypeStruct((b, _A * _ADIM), jnp.float32),
        compiler_params=pltpu.CompilerParams(
            dimension_semantics=("parallel",)),
    )(
        aid, signals, act2d, W0, a0, bias0[None, :],
        W1, a1, bias1[None, :],
        in_proj_w, in_proj_b[None, :],
        out_proj_w, out_proj_b[None, :],
        Wm, bm[None, :], Wa, ba[None, :],
    )
